# baseline (speedup 1.0000x reference)
import functools

import jax
import jax.numpy as jnp
from jax import lax
from jax.experimental import pallas as pl
from jax.experimental.pallas import tpu as pltpu
from jax.experimental.pallas import tpu_sc as plsc

N = 320000
H = 128
G = 1024

_BLK = 512

_SEG0_SC = 156
_SEG0_LEN = _SEG0_SC * 1024
_SEG1_LEN = N - _SEG0_LEN


def _gate_body(emb_ref, feat_ref, wg1_ref, wg2_ref, bg_ref, wt_ref, bt_ref,
               out_ref):
    emb = emb_ref[...]
    feat = feat_ref[...]
    z = (jnp.dot(emb, wg1_ref[...], preferred_element_type=jnp.float32)
         + jnp.dot(feat, wg2_ref[...], preferred_element_type=jnp.float32)
         + bg_ref[...])
    gate = jax.nn.sigmoid(z)
    t = jnp.tanh(
        jnp.dot(emb, wt_ref[...], preferred_element_type=jnp.float32)
        + bt_ref[...])
    out_ref[...] = gate * t


def _gate_call(emb, feat, wg1t, wg2t, bg2, wtt, bt2, blk0, seg_len):
    nblocks = seg_len // _BLK
    row_spec = pl.BlockSpec((_BLK, H), lambda i: (i + blk0, 0))
    out_spec = pl.BlockSpec((_BLK, H), lambda i: (i, 0))
    w_spec = pl.BlockSpec((H, H), lambda i: (0, 0))
    b_spec = pl.BlockSpec((1, H), lambda i: (0, 0))
    return pl.pallas_call(
        _gate_body,
        grid=(nblocks,),
        in_specs=[row_spec, row_spec, w_spec, w_spec, b_spec, w_spec, b_spec],
        out_specs=out_spec,
        out_shape=jax.ShapeDtypeStruct((seg_len, H), jnp.float32),
    )(emb, feat, wg1t, wg2t, bg2, wtt, bt2)


_RP = 2504
_SC_FULL = N // 1024
_GPS = G // 16
_Q = 256


def _make_segsum_body(sc0, scnt, has_tail):
    q, r = divmod(scnt, 32)

    def body(gated, ids, zeros64, out, idx_v, rows0, rows1, stage_v, acc,
             sem0, sem1):
        c = lax.axis_index("c")
        s = lax.axis_index("s")
        wid = s * 2 + c
        bufs = (rows0, rows1)
        sems = (sem0, sem1)

        pltpu.sync_copy(zeros64, stage_v)
        pltpu.sync_copy(stage_v, acc.at[pl.ds(s * _GPS, _GPS)])
        plsc.subcore_barrier()

        base = wid * q + jnp.minimum(wid, r)
        cnt = q + (wid < r)

        def gather_start(node0, buf, sem):
            pltpu.make_async_copy(gated.at[pl.ds(node0, _Q)], buf,
                                  sem).start()

        def gather_wait(buf, sem):
            pltpu.make_async_copy(gated.at[pl.ds(0, _Q)], buf, sem).wait()

        gather_start(base * 1024, rows0, sem0)

        def chunk_body(j, carry):
            sc = base + j
            node0 = sc * 1024
            pltpu.sync_copy(ids.at[pl.ds((sc0 + sc) * 8, 8)], idx_v)
            for qq in range(4):
                buf, sem = bufs[qq % 2], sems[qq % 2]
                nbuf, nsem = bufs[(qq + 1) % 2], sems[(qq + 1) % 2]
                gather_wait(buf, sem)
                if qq < 3:
                    gather_start(node0 + (qq + 1) * _Q, nbuf, nsem)
                else:
                    @pl.when(j + 1 < cnt)
                    def _prefetch_next():
                        gather_start(node0 + 1024, nbuf, nsem)
                for h in range(2):
                    pltpu.sync_copy(buf.at[pl.ds(h * H, H)],
                                    acc.at[idx_v.at[2 * qq + h]], add=True)
            return carry

        lax.fori_loop(0, cnt, chunk_body, 0)

        if has_tail:
            @pl.when(wid == 31)
            def _tail():
                pltpu.sync_copy(ids.at[pl.ds(_SC_FULL * 8, 8)], idx_v)
                tail0 = scnt * 1024
                for qq in range(2):
                    pltpu.sync_copy(gated.at[pl.ds(tail0 + qq * _Q, _Q)],
                                    rows0)
                    for h in range(2):
                        pltpu.sync_copy(rows0.at[pl.ds(h * H, H)],
                                        acc.at[idx_v.at[2 * qq + h]],
                                        add=True)

        plsc.subcore_barrier()

        pltpu.sync_copy(acc.at[pl.ds(s * _GPS, _GPS)], stage_v)
        pltpu.sync_copy(stage_v, out.at[c].at[pl.ds(s * _GPS, _GPS)])

    return body


def _make_segsum_call(sc0, scnt, has_tail):
    return functools.partial(
        pl.kernel,
        out_type=jax.ShapeDtypeStruct((2, G, H), jnp.float32),
        mesh=plsc.VectorSubcoreMesh(core_axis_name="c", subcore_axis_name="s"),
        scratch_types=[
            pltpu.VMEM((8, H), jnp.int32),
            pltpu.VMEM((_Q, H), jnp.float32),
            pltpu.VMEM((_Q, H), jnp.float32),
            pltpu.VMEM((_GPS, H), jnp.float32),
            pltpu.VMEM_SHARED((G, H), jnp.float32),
            pltpu.SemaphoreType.DMA,
            pltpu.SemaphoreType.DMA,
        ],
    )(_make_segsum_body(sc0, scnt, has_tail))


_segsum_seg0 = _make_segsum_call(0, _SEG0_SC, False)
_segsum_seg1 = _make_segsum_call(_SEG0_SC, _SC_FULL - _SEG0_SC, True)


def _combine_body(p0_ref, p1_ref, wo_ref, bo_ref, pred_ref, repr_ref):
    grepr = p0_ref[0] + p0_ref[1] + p1_ref[0] + p1_ref[1]
    repr_ref[...] = grepr
    pred_ref[...] = (jnp.sum(grepr * wo_ref[...], axis=1, keepdims=True)
                     + bo_ref[...])


def _combine_call(partials0, partials1, wo, bo2):
    return pl.pallas_call(
        _combine_body,
        out_shape=(
            jax.ShapeDtypeStruct((G, 1), jnp.float32),
            jax.ShapeDtypeStruct((G, H), jnp.float32),
        ),
    )(partials0, partials1, wo, bo2)


def kernel(node_embeddings, initial_features, graph_nodes_list, num_graphs,
           Wg, bg, Wt, bt, Wo, bo):
    wg1t = Wg[:, :H].T
    wg2t = Wg[:, H:].T
    wtt = Wt.T
    bg2 = bg.reshape(1, H)
    bt2 = bt.reshape(1, H)
    ids2d = jnp.concatenate(
        [graph_nodes_list,
         jnp.zeros((_RP * H - N,), jnp.int32)]).reshape(_RP, H)
    zeros64 = jnp.zeros((_GPS, H), jnp.float32)

    gated0 = _gate_call(node_embeddings, initial_features, wg1t, wg2t, bg2,
                        wtt, bt2, 0, _SEG0_LEN)
    gated1 = _gate_call(node_embeddings, initial_features, wg1t, wg2t, bg2,
                        wtt, bt2, _SEG0_LEN // _BLK, _SEG1_LEN)
    partials0 = _segsum_seg0(gated0, ids2d, zeros64)
    partials1 = _segsum_seg1(gated1, ids2d, zeros64)
    pred, graph_repr = _combine_call(partials0, partials1, Wo,
                                     bo.reshape(1, 1))
    return pred.reshape(G), graph_repr


# speedup vs baseline: 2.0431x; 2.0431x over previous
import functools

import jax
import jax.numpy as jnp
from jax import lax
from jax.experimental import pallas as pl
from jax.experimental.pallas import tpu as pltpu
from jax.experimental.pallas import tpu_sc as plsc

N = 320000
H = 128
G = 1024

_BLK = 4096

_SEG0_SC = 156
_SEG0_LEN = _SEG0_SC * 1024
_TAIL_LEN = N - 2 * _SEG0_LEN


def _gate_body(emb_ref, feat_ref, wg1_ref, wg2_ref, bg_ref, wt_ref, bt_ref,
               out_ref):
    emb = emb_ref[...]
    feat = feat_ref[...]
    z = (jnp.dot(emb, wg1_ref[...], preferred_element_type=jnp.float32)
         + jnp.dot(feat, wg2_ref[...], preferred_element_type=jnp.float32)
         + bg_ref[...])
    gate = jax.nn.sigmoid(z)
    t = jnp.tanh(
        jnp.dot(emb, wt_ref[...], preferred_element_type=jnp.float32)
        + bt_ref[...])
    out_ref[...] = gate * t


def _gate_call(emb, feat, wg1t, wg2t, bg2, wtt, bt2, row0, seg_len, blk):
    nblocks = seg_len // blk
    blk0 = row0 // blk
    row_spec = pl.BlockSpec((blk, H), lambda i: (i + blk0, 0))
    out_spec = pl.BlockSpec((blk, H), lambda i: (i, 0))
    w_spec = pl.BlockSpec((H, H), lambda i: (0, 0))
    b_spec = pl.BlockSpec((1, H), lambda i: (0, 0))
    return pl.pallas_call(
        _gate_body,
        grid=(nblocks,),
        in_specs=[row_spec, row_spec, w_spec, w_spec, b_spec, w_spec, b_spec],
        out_specs=out_spec,
        out_shape=jax.ShapeDtypeStruct((seg_len, H), jnp.float32),
    )(emb, feat, wg1t, wg2t, bg2, wtt, bt2)


_BLK_T = 512


_RP = 2504
_SC_FULL = N // 1024
_GPS = G // 16
_Q = 256


def _make_segsum_body(sc0, scnt, has_tail):
    q, r = divmod(scnt, 32)

    def body(gated, *rest):
        if has_tail:
            gated_tail, ids, zeros64, out, idx_v, rows0, rows1, stage_v, \
                acc, sem0, sem1 = rest
        else:
            ids, zeros64, out, idx_v, rows0, rows1, stage_v, \
                acc, sem0, sem1 = rest
        c = lax.axis_index("c")
        s = lax.axis_index("s")
        wid = s * 2 + c
        bufs = (rows0, rows1)
        sems = (sem0, sem1)

        pltpu.sync_copy(zeros64, stage_v)
        pltpu.sync_copy(stage_v, acc.at[pl.ds(s * _GPS, _GPS)])
        plsc.subcore_barrier()

        base = wid * q + jnp.minimum(wid, r)
        cnt = q + (wid < r)

        def gather_start(node0, buf, sem):
            pltpu.make_async_copy(gated.at[pl.ds(node0, _Q)], buf,
                                  sem).start()

        def gather_wait(buf, sem):
            pltpu.make_async_copy(gated.at[pl.ds(0, _Q)], buf, sem).wait()

        gather_start(base * 1024, rows0, sem0)

        def chunk_body(j, carry):
            sc = base + j
            node0 = sc * 1024
            pltpu.sync_copy(ids.at[pl.ds((sc0 + sc) * 8, 8)], idx_v)
            for qq in range(4):
                buf, sem = bufs[qq % 2], sems[qq % 2]
                nbuf, nsem = bufs[(qq + 1) % 2], sems[(qq + 1) % 2]
                gather_wait(buf, sem)
                if qq < 3:
                    gather_start(node0 + (qq + 1) * _Q, nbuf, nsem)
                else:
                    @pl.when(j + 1 < cnt)
                    def _prefetch_next():
                        gather_start(node0 + 1024, nbuf, nsem)
                for h in range(2):
                    pltpu.sync_copy(buf.at[pl.ds(h * H, H)],
                                    acc.at[idx_v.at[2 * qq + h]], add=True)
            return carry

        lax.fori_loop(0, cnt, chunk_body, 0)

        if has_tail:
            @pl.when(wid == 31)
            def _tail():
                pltpu.sync_copy(ids.at[pl.ds(_SC_FULL * 8, 8)], idx_v)
                for qq in range(2):
                    pltpu.sync_copy(gated_tail.at[pl.ds(qq * _Q, _Q)],
                                    rows0)
                    for h in range(2):
                        pltpu.sync_copy(rows0.at[pl.ds(h * H, H)],
                                        acc.at[idx_v.at[2 * qq + h]],
                                        add=True)

        plsc.subcore_barrier()

        pltpu.sync_copy(acc.at[pl.ds(s * _GPS, _GPS)], stage_v)
        pltpu.sync_copy(stage_v, out.at[c].at[pl.ds(s * _GPS, _GPS)])

    return body


def _make_segsum_call(sc0, scnt, has_tail):
    return functools.partial(
        pl.kernel,
        out_type=jax.ShapeDtypeStruct((2, G, H), jnp.float32),
        mesh=plsc.VectorSubcoreMesh(core_axis_name="c", subcore_axis_name="s"),
        scratch_types=[
            pltpu.VMEM((8, H), jnp.int32),
            pltpu.VMEM((_Q, H), jnp.float32),
            pltpu.VMEM((_Q, H), jnp.float32),
            pltpu.VMEM((_GPS, H), jnp.float32),
            pltpu.VMEM_SHARED((G, H), jnp.float32),
            pltpu.SemaphoreType.DMA,
            pltpu.SemaphoreType.DMA,
        ],
    )(_make_segsum_body(sc0, scnt, has_tail))


_segsum_seg0 = _make_segsum_call(0, _SEG0_SC, False)
_segsum_seg1 = _make_segsum_call(_SEG0_SC, _SC_FULL - _SEG0_SC, True)


def _combine_body(p0_ref, p1_ref, wo_ref, bo_ref, pred_ref, repr_ref):
    grepr = p0_ref[0] + p0_ref[1] + p1_ref[0] + p1_ref[1]
    repr_ref[...] = grepr
    pred_ref[...] = (jnp.sum(grepr * wo_ref[...], axis=1, keepdims=True)
                     + bo_ref[...])


def _combine_call(partials0, partials1, wo, bo2):
    return pl.pallas_call(
        _combine_body,
        out_shape=(
            jax.ShapeDtypeStruct((G, 1), jnp.float32),
            jax.ShapeDtypeStruct((G, H), jnp.float32),
        ),
    )(partials0, partials1, wo, bo2)


def kernel(node_embeddings, initial_features, graph_nodes_list, num_graphs,
           Wg, bg, Wt, bt, Wo, bo):
    wg1t = Wg[:, :H].T
    wg2t = Wg[:, H:].T
    wtt = Wt.T
    bg2 = bg.reshape(1, H)
    bt2 = bt.reshape(1, H)
    ids2d = jnp.concatenate(
        [graph_nodes_list,
         jnp.zeros((_RP * H - N,), jnp.int32)]).reshape(_RP, H)
    zeros64 = jnp.zeros((_GPS, H), jnp.float32)

    gated0 = _gate_call(node_embeddings, initial_features, wg1t, wg2t, bg2,
                        wtt, bt2, 0, _SEG0_LEN, _BLK)
    gated1 = _gate_call(node_embeddings, initial_features, wg1t, wg2t, bg2,
                        wtt, bt2, _SEG0_LEN, _SEG0_LEN, _BLK)
    gated_t = _gate_call(node_embeddings, initial_features, wg1t, wg2t, bg2,
                         wtt, bt2, 2 * _SEG0_LEN, _TAIL_LEN, _BLK_T)
    partials0 = _segsum_seg0(gated0, ids2d, zeros64)
    partials1 = _segsum_seg1(gated1, gated_t, ids2d, zeros64)
    pred, graph_repr = _combine_call(partials0, partials1, Wo,
                                     bo.reshape(1, 1))
    return pred.reshape(G), graph_repr


# speedup vs baseline: 2.0717x; 1.0140x over previous
import functools

import jax
import jax.numpy as jnp
from jax import lax
from jax.experimental import pallas as pl
from jax.experimental.pallas import tpu as pltpu
from jax.experimental.pallas import tpu_sc as plsc

N = 320000
H = 128
G = 1024

_BLK = 4096
_BLK_T = 512

_SEG_SC = 104
_SEG_LEN = _SEG_SC * 1024
_TAIL_LEN = N - 3 * _SEG_LEN


def _gate_body(emb_ref, feat_ref, wg_ref, bg_ref, wt_ref, bt_ref, out_ref):
    emb = emb_ref[...]
    feat = feat_ref[...]
    wg = wg_ref[...]
    cdims = (((1,), (1,)), ((), ()))
    z = (lax.dot_general(emb, wg[:, :H], cdims,
                         preferred_element_type=jnp.float32)
         + lax.dot_general(feat, wg[:, H:], cdims,
                           preferred_element_type=jnp.float32)
         + bg_ref[...])
    gate = jax.nn.sigmoid(z)
    t = jnp.tanh(
        lax.dot_general(emb, wt_ref[...], cdims,
                        preferred_element_type=jnp.float32)
        + bt_ref[...])
    out_ref[...] = gate * t


def _gate_call(emb, feat, wg, bg2, wt, bt2, row0, seg_len, blk):
    nblocks = seg_len // blk
    blk0 = row0 // blk
    row_spec = pl.BlockSpec((blk, H), lambda i: (i + blk0, 0))
    out_spec = pl.BlockSpec((blk, H), lambda i: (i, 0))
    wg_spec = pl.BlockSpec((H, 2 * H), lambda i: (0, 0))
    wt_spec = pl.BlockSpec((H, H), lambda i: (0, 0))
    b_spec = pl.BlockSpec((1, H), lambda i: (0, 0))
    return pl.pallas_call(
        _gate_body,
        grid=(nblocks,),
        in_specs=[row_spec, row_spec, wg_spec, b_spec, wt_spec, b_spec],
        out_specs=out_spec,
        out_shape=jax.ShapeDtypeStruct((seg_len, H), jnp.float32),
    )(emb, feat, wg, bg2, wt, bt2)


_RP = 2504
_SC_FULL = N // 1024
_GPS = G // 16
_Q = 256


def _make_segsum_body(sc0, scnt, has_tail):
    q, r = divmod(scnt, 32)

    def body(gated, *rest):
        if has_tail:
            gated_tail, ids, zeros64, out, idx_v, rows0, rows1, stage_v, \
                acc, sem0, sem1 = rest
        else:
            ids, zeros64, out, idx_v, rows0, rows1, stage_v, \
                acc, sem0, sem1 = rest
        c = lax.axis_index("c")
        s = lax.axis_index("s")
        wid = s * 2 + c
        bufs = (rows0, rows1)
        sems = (sem0, sem1)

        pltpu.sync_copy(zeros64, stage_v)
        pltpu.sync_copy(stage_v, acc.at[pl.ds(s * _GPS, _GPS)])
        plsc.subcore_barrier()

        base = wid * q + jnp.minimum(wid, r)
        cnt = q + (wid < r)

        def gather_start(node0, buf, sem):
            pltpu.make_async_copy(gated.at[pl.ds(node0, _Q)], buf,
                                  sem).start()

        def gather_wait(buf, sem):
            pltpu.make_async_copy(gated.at[pl.ds(0, _Q)], buf, sem).wait()

        gather_start(base * 1024, rows0, sem0)

        def chunk_body(j, carry):
            sc = base + j
            node0 = sc * 1024
            pltpu.sync_copy(ids.at[pl.ds((sc0 + sc) * 8, 8)], idx_v)
            for qq in range(4):
                buf, sem = bufs[qq % 2], sems[qq % 2]
                nbuf, nsem = bufs[(qq + 1) % 2], sems[(qq + 1) % 2]
                gather_wait(buf, sem)
                if qq < 3:
                    gather_start(node0 + (qq + 1) * _Q, nbuf, nsem)
                else:
                    @pl.when(j + 1 < cnt)
                    def _prefetch_next():
                        gather_start(node0 + 1024, nbuf, nsem)
                for h in range(2):
                    pltpu.sync_copy(buf.at[pl.ds(h * H, H)],
                                    acc.at[idx_v.at[2 * qq + h]], add=True)
            return carry

        lax.fori_loop(0, cnt, chunk_body, 0)

        if has_tail:
            @pl.when(wid == 31)
            def _tail():
                pltpu.sync_copy(ids.at[pl.ds(_SC_FULL * 8, 8)], idx_v)
                for qq in range(2):
                    pltpu.sync_copy(gated_tail.at[pl.ds(qq * _Q, _Q)],
                                    rows0)
                    for h in range(2):
                        pltpu.sync_copy(rows0.at[pl.ds(h * H, H)],
                                        acc.at[idx_v.at[2 * qq + h]],
                                        add=True)

        plsc.subcore_barrier()

        pltpu.sync_copy(acc.at[pl.ds(s * _GPS, _GPS)], stage_v)
        pltpu.sync_copy(stage_v, out.at[c].at[pl.ds(s * _GPS, _GPS)])

    return body


def _make_segsum_call(sc0, scnt, has_tail):
    return functools.partial(
        pl.kernel,
        out_type=jax.ShapeDtypeStruct((2, G, H), jnp.float32),
        mesh=plsc.VectorSubcoreMesh(core_axis_name="c", subcore_axis_name="s"),
        scratch_types=[
            pltpu.VMEM((8, H), jnp.int32),
            pltpu.VMEM((_Q, H), jnp.float32),
            pltpu.VMEM((_Q, H), jnp.float32),
            pltpu.VMEM((_GPS, H), jnp.float32),
            pltpu.VMEM_SHARED((G, H), jnp.float32),
            pltpu.SemaphoreType.DMA,
            pltpu.SemaphoreType.DMA,
        ],
    )(_make_segsum_body(sc0, scnt, has_tail))


_segsum_seg0 = _make_segsum_call(0, _SEG_SC, False)
_segsum_seg1 = _make_segsum_call(_SEG_SC, _SEG_SC, False)
_segsum_seg2 = _make_segsum_call(2 * _SEG_SC, _SEG_SC, True)


def _combine_body(p0_ref, p1_ref, p2_ref, wo_ref, bo_ref, pred_ref,
                  repr_ref):
    grepr = (p0_ref[0] + p0_ref[1] + p1_ref[0] + p1_ref[1]
             + p2_ref[0] + p2_ref[1])
    repr_ref[...] = grepr
    pred_ref[...] = (jnp.sum(grepr * wo_ref[...], axis=1, keepdims=True)
                     + bo_ref[...])


def _combine_call(partials, wo, bo2):
    return pl.pallas_call(
        _combine_body,
        out_shape=(
            jax.ShapeDtypeStruct((G, 1), jnp.float32),
            jax.ShapeDtypeStruct((G, H), jnp.float32),
        ),
    )(*partials, wo, bo2)


def kernel(node_embeddings, initial_features, graph_nodes_list, num_graphs,
           Wg, bg, Wt, bt, Wo, bo):
    bg2 = bg.reshape(1, H)
    bt2 = bt.reshape(1, H)
    ids2d = jnp.concatenate(
        [graph_nodes_list,
         jnp.zeros((_RP * H - N,), jnp.int32)]).reshape(_RP, H)
    zeros64 = jnp.zeros((_GPS, H), jnp.float32)

    gated = [
        _gate_call(node_embeddings, initial_features, Wg, bg2, Wt, bt2,
                   k * _SEG_LEN, _SEG_LEN, _BLK)
        for k in range(3)
    ]
    gated_t = _gate_call(node_embeddings, initial_features, Wg, bg2, Wt, bt2,
                         3 * _SEG_LEN, _TAIL_LEN, _BLK_T)
    partials = [
        _segsum_seg0(gated[0], ids2d, zeros64),
        _segsum_seg1(gated[1], ids2d, zeros64),
        _segsum_seg2(gated[2], gated_t, ids2d, zeros64),
    ]
    pred, graph_repr = _combine_call(partials, Wo, bo.reshape(1, 1))
    return pred.reshape(G), graph_repr
